# Initial kernel scaffold; baseline (speedup 1.0000x reference)
import jax
import jax.numpy as jnp
from jax.experimental import pallas as pl


def kernel(Z, z_to_index):
    raise NotImplementedError("write your pallas kernel here")


# speedup vs baseline: 5.6958x; 5.6958x over previous
import functools

import jax
import jax.numpy as jnp
from jax import lax
from jax.experimental import pallas as pl
from jax.experimental.pallas import tpu as pltpu
from jax.experimental.pallas import tpu_sc as plsc

N = 100000
D = 119
DP = 128
L = 16
NC, NS = 2, 16
NW = NC * NS
GROUPS = 16
BLOCK = GROUPS * L
BT = BLOCK // 8

NBLK = N // BLOCK
TFULL = NBLK // NW
XBLK = NBLK - TFULL * NW
TAIL0 = NBLK * BLOCK
TAILG = (N - TAIL0) // L


@functools.cache
def _build():
    mesh = plsc.VectorSubcoreMesh(
        core_axis_name="c", subcore_axis_name="s", num_cores=NC, num_subcores=NS
    )

    @functools.partial(
        pl.kernel,
        out_type=jax.ShapeDtypeStruct((N // 8, 8, DP), jnp.int32),
        mesh=mesh,
        compiler_params=pltpu.CompilerParams(needs_layout_passes=False),
        scratch_types=[
            pltpu.VMEM((128,), jnp.int32),
            pltpu.VMEM((BLOCK,), jnp.int32),
            pltpu.VMEM((BT, 8, DP), jnp.int32),
        ],
    )
    def onehot_sc(z_hbm, table_hbm, out_hbm, table_v, z_v, buf):
        wid = lax.axis_index("s") * NC + lax.axis_index("c")
        iota = lax.iota(jnp.int32, L)
        ones = jnp.ones((L,), jnp.int32)
        zeros = jnp.zeros((L,), jnp.int32)
        sub = jnp.bitwise_and(iota, 7)
        tof = lax.shift_right_logical(iota, 3)

        pltpu.sync_copy(table_hbm, table_v)

        def _zero_tile(t, carry):
            for s in range(8):
                for g in range(8):
                    buf[t, s, pl.ds(g * L, L)] = zeros
            return carry

        lax.fori_loop(0, BT, _zero_tile, 0)

        def scatter_vals(ngroups, vals):
            for g in range(ngroups):
                zv = z_v[pl.ds(g * L, L)]
                idx = plsc.load_gather(table_v, [zv])
                plsc.store_scatter(buf, [2 * g + tof, sub, idx], vals)

        def emit(row0, ngroups):
            nrows = ngroups * L
            pltpu.sync_copy(z_hbm.at[pl.ds(row0, nrows)], z_v.at[pl.ds(0, nrows)])
            scatter_vals(ngroups, ones)
            pltpu.sync_copy(
                buf.at[pl.ds(0, nrows // 8)],
                out_hbm.at[pl.ds(row0 // 8, nrows // 8)],
            )
            scatter_vals(ngroups, zeros)

        def round_body(t, carry):
            emit((t * NW + wid) * BLOCK, GROUPS)
            return carry

        lax.fori_loop(0, TFULL, round_body, 0)

        @pl.when(wid < XBLK)
        def _():
            emit((TFULL * NW + wid) * BLOCK, GROUPS)

        @pl.when(wid < TAILG)
        def _():
            emit(TAIL0 + wid * L, 1)

    return onehot_sc


def kernel(Z, z_to_index):
    zi = Z.astype(jnp.int32)
    table = jnp.zeros((128,), jnp.int32).at[:D].set(z_to_index.astype(jnp.int32))
    out3 = _build()(zi, table)
    return out3.reshape(N, DP)[:, :D]
